# Initial kernel scaffold; baseline (speedup 1.0000x reference)
import jax
import jax.numpy as jnp
from jax.experimental import pallas as pl


def kernel(all_types, all_prices, all_groups, type_table, price_table):
    raise NotImplementedError("write your pallas kernel here")


# speedup vs baseline: 1.2307x; 1.2307x over previous
import functools

import jax
import jax.numpy as jnp
from jax import lax
from jax.experimental import pallas as pl
from jax.experimental.pallas import tpu as pltpu
from jax.experimental.pallas import tpu_sc as plsc

BATCH = 16384
TYPE_DIM = 64
PRICE_DIM = 32
GROUP_DIM = 32

NUM_CORES = 2
NUM_WORKERS = 32
ROWS_PER_WORKER = BATCH // NUM_WORKERS
CHUNK = 128
NUM_ROUNDS = ROWS_PER_WORKER // CHUNK


def _merge_row(trows, prows, grows, k):
    trows[k, pl.ds(64, 16)] = prows[k, pl.ds(0, 16)]
    trows[k, pl.ds(80, 16)] = prows[k, pl.ds(16, 16)]
    trows[k, pl.ds(96, 16)] = grows[k, pl.ds(0, 16)]
    trows[k, pl.ds(112, 16)] = grows[k, pl.ds(16, 16)]


@functools.partial(
    pl.kernel,
    mesh=plsc.VectorSubcoreMesh(core_axis_name="c", subcore_axis_name="s"),
    out_type=jax.ShapeDtypeStruct((BATCH, 128), jnp.float32),
    scratch_types=[
        pltpu.VMEM((NUM_ROUNDS, CHUNK), jnp.int32),
        pltpu.VMEM((NUM_ROUNDS, CHUNK), jnp.int32),
        pltpu.VMEM((NUM_ROUNDS, CHUNK), jnp.int32),
        pltpu.VMEM((CHUNK, 128), jnp.float32),
        pltpu.VMEM((CHUNK, 128), jnp.float32),
        pltpu.VMEM((CHUNK, 128), jnp.float32),
        pltpu.SemaphoreType.DMA,
        pltpu.SemaphoreType.DMA,
        pltpu.SemaphoreType.DMA,
    ],
)
def _sc_call(types_hbm, prices_hbm, iota_hbm, ttab_hbm, ptab_hbm, gtab_hbm,
             out_hbm, idx_t, idx_p, idx_g, trows, prows, grows,
             sem_t, sem_p, sem_g):
    wid = lax.axis_index("s") * NUM_CORES + lax.axis_index("c")
    base = wid * ROWS_PER_WORKER
    row0 = wid * NUM_ROUNDS

    pltpu.sync_copy(types_hbm.at[pl.ds(row0, NUM_ROUNDS)], idx_t)
    pltpu.sync_copy(prices_hbm.at[pl.ds(row0, NUM_ROUNDS)], idx_p)
    pltpu.sync_copy(iota_hbm.at[pl.ds(row0, NUM_ROUNDS)], idx_g)

    for j in range(NUM_ROUNDS):
        tcp = pltpu.async_copy(ttab_hbm.at[idx_t.at[j]], trows, sem_t)
        pcp = pltpu.async_copy(ptab_hbm.at[idx_p.at[j]], prows, sem_p)
        gcp = pltpu.async_copy(gtab_hbm.at[idx_g.at[j]], grows, sem_g)
        tcp.wait()
        pcp.wait()
        gcp.wait()

        def body(k, carry):
            _merge_row(trows, prows, grows, k)
            return carry

        lax.fori_loop(0, CHUNK, body, 0)
        pltpu.sync_copy(trows, out_hbm.at[pl.ds(base + j * CHUNK, CHUNK)])


def kernel(all_types, all_prices, all_groups, type_table, price_table):
    ttab = jnp.pad(type_table, ((0, 0), (0, 128 - TYPE_DIM)))
    ptab = jnp.pad(price_table, ((0, 0), (0, 128 - PRICE_DIM)))
    gtab = jnp.pad(all_groups, ((0, 0), (0, 128 - GROUP_DIM)))
    types2d = all_types.astype(jnp.int32).reshape(BATCH // CHUNK, CHUNK)
    prices2d = all_prices.astype(jnp.int32).reshape(BATCH // CHUNK, CHUNK)
    iota2d = jnp.arange(BATCH, dtype=jnp.int32).reshape(BATCH // CHUNK, CHUNK)
    return _sc_call(types2d, prices2d, iota2d, ttab, ptab, gtab)
